# Initial kernel scaffold; baseline (speedup 1.0000x reference)
import jax
import jax.numpy as jnp
from jax.experimental import pallas as pl


def kernel(inputs, targets, vocab_batches, emb1, emb2):
    raise NotImplementedError("write your pallas kernel here")


# speedup vs baseline: 3.1613x; 3.1613x over previous
import functools

import jax
import jax.numpy as jnp
from jax import lax
from jax.experimental import pallas as pl
from jax.experimental.pallas import tpu as pltpu
from jax.experimental.pallas import tpu_sc as plsc

_NC = 2
_NS = 16
_NW = _NC * _NS
_CH = 128


def _make_gather(V, D, N1, N2):
  n1 = N1 // _NW
  n2 = N2 // _NW
  assert n1 % _CH == 0 and n2 % _CH == 0

  mesh = plsc.VectorSubcoreMesh(core_axis_name="c", subcore_axis_name="s")

  @functools.partial(
      pl.kernel,
      out_type=[
          jax.ShapeDtypeStruct((N1, D), jnp.float32),
          jax.ShapeDtypeStruct((N2, D), jnp.float32),
      ],
      mesh=mesh,
      compiler_params=pltpu.CompilerParams(use_tc_tiling_on_sc=False),
      scratch_types=[
          pltpu.VMEM((n1,), jnp.int32),
          pltpu.VMEM((n2,), jnp.int32),
          pltpu.VMEM((_CH, D), jnp.float32),
          pltpu.SemaphoreType.DMA,
      ],
  )
  def gather(emb1, emb2, idx1, idx2, out1, out2, idx1_v, idx2_v, rows_v, sem):
    wid = lax.axis_index("s") * _NC + lax.axis_index("c")
    b1 = wid * n1
    b2 = wid * n2
    pltpu.sync_copy(idx1.at[pl.ds(b1, n1)], idx1_v)
    pltpu.sync_copy(idx2.at[pl.ds(b2, n2)], idx2_v)

    def body1(i, _):
      off = pl.multiple_of(i * _CH, _CH)
      pltpu.async_copy(emb1.at[idx1_v.at[pl.ds(off, _CH)]], rows_v, sem).wait()
      pltpu.sync_copy(rows_v, out1.at[pl.ds(b1 + off, _CH)])
      return 0

    lax.fori_loop(0, n1 // _CH, body1, 0)

    def body2(i, _):
      off = pl.multiple_of(i * _CH, _CH)
      pltpu.async_copy(emb2.at[idx2_v.at[pl.ds(off, _CH)]], rows_v, sem).wait()
      pltpu.sync_copy(rows_v, out2.at[pl.ds(b2 + off, _CH)])
      return 0

    lax.fori_loop(0, n2 // _CH, body2, 0)

  return gather


def _score_body(C, K1, D, BB, SUB, eI_ref, eTV_ref, out_ref):
  @pl.when(pl.program_id(0) == 0)
  def _init():
    out_ref[...] = jnp.zeros_like(out_ref)

  def chunk(j, acc):
    tv = eTV_ref[pl.ds(j * SUB * K1, SUB * K1), :]
    ii = eI_ref[pl.ds(j * SUB * C, SUB * C), :]
    tv3 = tv.reshape(SUB, K1, D)
    ii3 = ii.reshape(SUB, C, D)
    n = lax.dot_general(
        tv3, ii3, (((2,), (2,)), ((0,), (0,))),
        preferred_element_type=jnp.float32)
    row = lax.broadcasted_iota(jnp.int32, (SUB, K1, C), 1)
    e = jnp.where(row > 0, jnp.exp(n), 0.0)
    colsum = jnp.sum(e, axis=1)
    acc_n = jnp.sum(jnp.log(colsum))
    acc_s = jnp.sum(jnp.where(row == 0, n, 0.0))
    return acc + (acc_n - acc_s)

  p = lax.fori_loop(0, BB // SUB, chunk, jnp.float32(0.0))
  out_ref[...] += jnp.reshape(p, (1, 1))


def kernel(inputs, targets, vocab_batches, emb1, emb2):
  B, C = inputs.shape
  K = vocab_batches.shape[1]
  V, D = emb1.shape
  K1 = K + 1

  idx1 = inputs.astype(jnp.int32).reshape(-1)
  idx2 = jnp.concatenate(
      [targets.astype(jnp.int32), vocab_batches.astype(jnp.int32)],
      axis=1).reshape(-1)

  eI, eTV = _make_gather(V, D, B * C, B * K1)(emb1, emb2, idx1, idx2)

  BB = 256
  SUB = 8

  partial = pl.pallas_call(
      functools.partial(_score_body, C, K1, D, BB, SUB),
      grid=(B // BB,),
      in_specs=[
          pl.BlockSpec((BB * C, D), lambda i: (i, 0)),
          pl.BlockSpec((BB * K1, D), lambda i: (i, 0)),
      ],
      out_specs=pl.BlockSpec((1, 1), lambda i: (0, 0)),
      out_shape=jax.ShapeDtypeStruct((1, 1), jnp.float32),
  )(eI, eTV)

  return partial[0, 0] / jnp.float32(B * C)
